# Initial kernel scaffold; baseline (speedup 1.0000x reference)
import functools

import jax
import jax.numpy as jnp
from jax import lax
from jax.experimental import pallas as pl
from jax.experimental.pallas import tpu as pltpu
from jax.experimental.pallas import tpu_sc as plsc

_NC = 2
_NS = 16
_LANES = 16


def _sc_scatter(N, D, E, K):
    NW = _NC * _NS
    ept = E // NW
    nch = ept // K
    rpt = N // _NS

    mesh = plsc.VectorSubcoreMesh(core_axis_name="c", subcore_axis_name="s")

    @functools.partial(
        pl.kernel,
        out_type=[
            jax.ShapeDtypeStruct((_NC, N, D), jnp.float32),
            jax.ShapeDtypeStruct((_NC, N), jnp.float32),
        ],
        mesh=mesh,
        scratch_types=[
            pltpu.VMEM((K,), jnp.int32),
            pltpu.VMEM((K,), jnp.int32),
            pltpu.VMEM((K, D), jnp.float32),
            pltpu.VMEM((K, D), jnp.float32),
            pltpu.VMEM((K,), jnp.float32),
            pltpu.VMEM_SHARED((N, D), jnp.float32),
            pltpu.VMEM_SHARED((N,), jnp.float32),
            pltpu.SemaphoreType.DMA,
        ],
    )
    def sc_kernel(head_hbm, tail_hbm, node_hbm, rel_hbm, z2_hbm, z1_hbm,
                  agg_out, deg_out,
                  hidx, tidx, nrows, rrows, ones, agg_sh, deg_sh, sem):
        c = lax.axis_index("c")
        s = lax.axis_index("s")

        pltpu.sync_copy(z2_hbm.at[pl.ds(s * rpt, rpt)],
                        agg_sh.at[pl.ds(s * rpt, rpt)])

        @pl.when(s == 0)
        def _():
            pltpu.sync_copy(z1_hbm, deg_sh)

        @pl.loop(0, K // _LANES)
        def _(j):
            ones[pl.ds(j * _LANES, _LANES)] = jnp.full((_LANES,), 1.0, jnp.float32)

        plsc.subcore_barrier()

        ebase = (c * _NS + s) * ept

        @pl.loop(0, nch)
        def _(i):
            base = ebase + i * K
            pltpu.sync_copy(head_hbm.at[pl.ds(base, K)], hidx)
            pltpu.sync_copy(tail_hbm.at[pl.ds(base, K)], tidx)
            pltpu.async_copy(node_hbm.at[hidx], nrows, sem).wait()
            pltpu.sync_copy(rel_hbm.at[pl.ds(base, K)], rrows)

            @pl.loop(0, K)
            def _(r):
                for j in range(D // _LANES):
                    sl = pl.ds(j * _LANES, _LANES)
                    rrows[r, sl] = rrows[r, sl] + nrows[r, sl]

            pltpu.sync_copy(rrows, agg_sh.at[tidx], add=True)
            pltpu.sync_copy(ones, deg_sh.at[tidx], add=True)

        plsc.subcore_barrier()

        pltpu.sync_copy(agg_sh.at[pl.ds(s * rpt, rpt)],
                        agg_out.at[c, pl.ds(s * rpt, rpt)])

        @pl.when(s == 0)
        def _():
            pltpu.sync_copy(deg_sh, deg_out.at[c])

    return sc_kernel


def _tc_dense(N, D, B):
    def body(node_ref, aggp_ref, degp_ref, wm_ref, bm_ref, wu1_ref, wu2_ref,
             bu_ref, g_ref, b_ref, out_ref):
        node = node_ref[...]
        agg_pre = aggp_ref[0] + aggp_ref[1]
        deg = degp_ref[:, 0:1] + degp_ref[:, 1:2]

        agg = jnp.dot(agg_pre, wm_ref[...], preferred_element_type=jnp.float32)
        agg = agg + deg * bm_ref[...]
        agg = agg / jnp.maximum(deg, 1.0)

        upd = (jnp.dot(node, wu1_ref[...], preferred_element_type=jnp.float32)
               + jnp.dot(agg, wu2_ref[...], preferred_element_type=jnp.float32)
               + bu_ref[...])
        out = node + jax.nn.gelu(upd, approximate=False)

        mean = jnp.mean(out, axis=-1, keepdims=True)
        cent = out - mean
        var = jnp.mean(cent * cent, axis=-1, keepdims=True)
        normed = cent * lax.rsqrt(var + 1e-5)
        out_ref[...] = normed * g_ref[...] + b_ref[...]

    grid = (N // B,)
    return pl.pallas_call(
        body,
        grid=grid,
        in_specs=[
            pl.BlockSpec((B, D), lambda i: (i, 0)),
            pl.BlockSpec((_NC, B, D), lambda i: (0, i, 0)),
            pl.BlockSpec((B, _NC), lambda i: (i, 0)),
            pl.BlockSpec((D, D), lambda i: (0, 0)),
            pl.BlockSpec((1, D), lambda i: (0, 0)),
            pl.BlockSpec((D, D), lambda i: (0, 0)),
            pl.BlockSpec((D, D), lambda i: (0, 0)),
            pl.BlockSpec((1, D), lambda i: (0, 0)),
            pl.BlockSpec((1, D), lambda i: (0, 0)),
            pl.BlockSpec((1, D), lambda i: (0, 0)),
        ],
        out_specs=pl.BlockSpec((B, D), lambda i: (i, 0)),
        out_shape=jax.ShapeDtypeStruct((N, D), jnp.float32),
    )


def kernel(node_tokens, relation_tokens, edge_index, num_nodes,
           W_msg, b_msg, W_upd, b_upd, gamma, beta):
    N, D = node_tokens.shape
    E = relation_tokens.shape[0]
    K = 80
    assert E % (_NC * _NS * K) == 0 and N % _NS == 0 and D % _LANES == 0

    head = edge_index[0]
    tail = edge_index[1]
    z2 = jnp.zeros((N, D), jnp.float32)
    z1 = jnp.zeros((N,), jnp.float32)

    agg_parts, deg_parts = _sc_scatter(N, D, E, K)(
        head, tail, node_tokens, relation_tokens, z2, z1)

    deg_t = deg_parts.T
    wm_t = W_msg.T
    wu1_t = W_upd[:, :D].T
    wu2_t = W_upd[:, D:].T

    B = 2000
    out = _tc_dense(N, D, B)(
        node_tokens, agg_parts, deg_t, wm_t,
        b_msg.reshape(1, D), wu1_t, wu2_t,
        b_upd.reshape(1, D), gamma.reshape(1, D), beta.reshape(1, D))
    return out


# speedup vs baseline: 3.9846x; 3.9846x over previous
import functools

import jax
import jax.numpy as jnp
from jax import lax
from jax.experimental import pallas as pl
from jax.experimental.pallas import tpu as pltpu
from jax.experimental.pallas import tpu_sc as plsc

_NC = 2
_NS = 16
_LANES = 16


def _sc_scatter(N, D, E, K):
    NW = _NC * _NS
    ept = E // NW
    nch = ept // K
    rpt = max(8, ((N // _NS + 7) // 8) * 8)
    while N % rpt != 0:
        rpt += 8
    nzt = N // rpt

    mesh = plsc.VectorSubcoreMesh(core_axis_name="c", subcore_axis_name="s")

    @functools.partial(
        pl.kernel,
        out_type=[
            jax.ShapeDtypeStruct((_NC, N, D), jnp.float32),
            jax.ShapeDtypeStruct((_NC, N), jnp.float32),
        ],
        mesh=mesh,
        scratch_types=[
            pltpu.VMEM((K,), jnp.int32),
            pltpu.VMEM((K,), jnp.int32),
            pltpu.VMEM((K, D), jnp.float32),
            pltpu.VMEM((K, D), jnp.float32),
            pltpu.VMEM((K,), jnp.float32),
            pltpu.VMEM_SHARED((N, D), jnp.float32),
            pltpu.VMEM_SHARED((N,), jnp.float32),
            pltpu.SemaphoreType.DMA,
        ],
    )
    def sc_kernel(head_hbm, tail_hbm, node_hbm, rel_hbm, z2_hbm, z1_hbm,
                  agg_out, deg_out,
                  hidx, tidx, nrows, rrows, ones, agg_sh, deg_sh, sem):
        c = lax.axis_index("c")
        s = lax.axis_index("s")

        @pl.when(s < nzt)
        def _():
            pltpu.sync_copy(z2_hbm.at[pl.ds(s * rpt, rpt)],
                            agg_sh.at[pl.ds(s * rpt, rpt)])

        @pl.when(s == 0)
        def _():
            pltpu.sync_copy(z1_hbm, deg_sh)

        @pl.loop(0, K // _LANES)
        def _(j):
            ones[pl.ds(j * _LANES, _LANES)] = jnp.full((_LANES,), 1.0, jnp.float32)

        plsc.subcore_barrier()

        ebase = (c * _NS + s) * ept

        @pl.loop(0, nch)
        def _(i):
            base = ebase + i * K
            pltpu.sync_copy(head_hbm.at[pl.ds(base, K)], hidx)
            pltpu.sync_copy(tail_hbm.at[pl.ds(base, K)], tidx)
            pltpu.async_copy(node_hbm.at[hidx], nrows, sem).wait()
            pltpu.sync_copy(rel_hbm.at[pl.ds(base, K)], rrows)

            @pl.loop(0, K)
            def _(r):
                for j in range(D // _LANES):
                    sl = pl.ds(j * _LANES, _LANES)
                    rrows[r, sl] = rrows[r, sl] + nrows[r, sl]

            pltpu.sync_copy(rrows, agg_sh.at[tidx], add=True)
            pltpu.sync_copy(ones, deg_sh.at[tidx], add=True)

        plsc.subcore_barrier()

        @pl.when(s < nzt)
        def _():
            pltpu.sync_copy(agg_sh.at[pl.ds(s * rpt, rpt)],
                            agg_out.at[c, pl.ds(s * rpt, rpt)])

        @pl.when(s == 0)
        def _():
            pltpu.sync_copy(deg_sh, deg_out.at[c])

    return sc_kernel


def _tc_dense(N, D, B):
    def body(node_ref, aggp_ref, degp_ref, wm_ref, bm_ref, wu1_ref, wu2_ref,
             bu_ref, g_ref, b_ref, out_ref):
        node = node_ref[...]
        agg_pre = aggp_ref[0] + aggp_ref[1]
        deg = degp_ref[:, 0:1] + degp_ref[:, 1:2]

        agg = jnp.dot(agg_pre, wm_ref[...], preferred_element_type=jnp.float32)
        agg = agg + deg * bm_ref[...]
        agg = agg / jnp.maximum(deg, 1.0)

        upd = (jnp.dot(node, wu1_ref[...], preferred_element_type=jnp.float32)
               + jnp.dot(agg, wu2_ref[...], preferred_element_type=jnp.float32)
               + bu_ref[...])
        gelu = 0.5 * upd * (1.0 + lax.erf(upd * 0.7071067811865476))
        out = node + gelu

        mean = jnp.mean(out, axis=-1, keepdims=True)
        cent = out - mean
        var = jnp.mean(cent * cent, axis=-1, keepdims=True)
        normed = cent * lax.rsqrt(var + 1e-5)
        out_ref[...] = normed * g_ref[...] + b_ref[...]

    grid = (N // B,)
    return pl.pallas_call(
        body,
        grid=grid,
        in_specs=[
            pl.BlockSpec((B, D), lambda i: (i, 0)),
            pl.BlockSpec((_NC, B, D), lambda i: (0, i, 0)),
            pl.BlockSpec((B, _NC), lambda i: (i, 0)),
            pl.BlockSpec((D, D), lambda i: (0, 0)),
            pl.BlockSpec((1, D), lambda i: (0, 0)),
            pl.BlockSpec((D, D), lambda i: (0, 0)),
            pl.BlockSpec((D, D), lambda i: (0, 0)),
            pl.BlockSpec((1, D), lambda i: (0, 0)),
            pl.BlockSpec((1, D), lambda i: (0, 0)),
            pl.BlockSpec((1, D), lambda i: (0, 0)),
        ],
        out_specs=pl.BlockSpec((B, D), lambda i: (i, 0)),
        out_shape=jax.ShapeDtypeStruct((N, D), jnp.float32),
    )


def kernel(node_tokens, relation_tokens, edge_index, num_nodes,
           W_msg, b_msg, W_upd, b_upd, gamma, beta):
    N, D = node_tokens.shape
    E = relation_tokens.shape[0]
    K = 80
    assert E % (_NC * _NS * K) == 0 and N % _NS == 0 and D % _LANES == 0

    head = edge_index[0]
    tail = edge_index[1]
    z2 = jnp.zeros((N, D), jnp.float32)
    z1 = jnp.zeros((N,), jnp.float32)

    agg_parts, deg_parts = _sc_scatter(N, D, E, K)(
        head, tail, node_tokens, relation_tokens, z2, z1)

    deg_t = deg_parts.T
    wm_t = W_msg.T
    wu1_t = W_upd[:, :D].T
    wu2_t = W_upd[:, D:].T

    B = 2000
    out = _tc_dense(N, D, B)(
        node_tokens, agg_parts, deg_t, wm_t,
        b_msg.reshape(1, D), wu1_t, wu2_t,
        b_upd.reshape(1, D), gamma.reshape(1, D), beta.reshape(1, D))
    return out


# speedup vs baseline: 6.2873x; 1.5779x over previous
import functools

import jax
import jax.numpy as jnp
from jax import lax
from jax.experimental import pallas as pl
from jax.experimental.pallas import tpu as pltpu
from jax.experimental.pallas import tpu_sc as plsc

_NC = 2
_NS = 16
_LANES = 16


def _sc_scatter(N, D, E, K, NB):
    NW = _NC * _NS
    ept = E // NW
    nch = ept // K
    rpt = max(8, ((N // _NS + 7) // 8) * 8)
    while N % rpt != 0:
        rpt += 8
    nzt = N // rpt

    mesh = plsc.VectorSubcoreMesh(core_axis_name="c", subcore_axis_name="s")

    @functools.partial(
        pl.kernel,
        out_type=[
            jax.ShapeDtypeStruct((_NC, N, D), jnp.float32),
            jax.ShapeDtypeStruct((_NC, N), jnp.float32),
        ],
        mesh=mesh,
        scratch_types=[
            [pltpu.VMEM((K,), jnp.int32)] * NB,
            [pltpu.VMEM((K,), jnp.int32)] * NB,
            [pltpu.VMEM((K, D), jnp.float32)] * NB,
            [pltpu.VMEM((K, D), jnp.float32)] * NB,
            pltpu.VMEM((K,), jnp.float32),
            pltpu.VMEM_SHARED((N, D), jnp.float32),
            pltpu.VMEM_SHARED((N,), jnp.float32),
            [pltpu.SemaphoreType.DMA] * NB,
            [pltpu.SemaphoreType.DMA] * NB,
            [pltpu.SemaphoreType.DMA] * NB,
        ],
    )
    def sc_kernel(head_hbm, tail_hbm, node_hbm, rel_hbm, z2_hbm, z1_hbm,
                  agg_out, deg_out,
                  hidx, tidx, nrows, rrows, ones, agg_sh, deg_sh,
                  sem_idx, sem_in, sem_out):
        c = lax.axis_index("c")
        s = lax.axis_index("s")

        @pl.when(s < nzt)
        def _():
            pltpu.sync_copy(z2_hbm.at[pl.ds(s * rpt, rpt)],
                            agg_sh.at[pl.ds(s * rpt, rpt)])

        @pl.when(s == 0)
        def _():
            pltpu.sync_copy(z1_hbm, deg_sh)

        @pl.loop(0, K // _LANES)
        def _(j):
            ones[pl.ds(j * _LANES, _LANES)] = jnp.full((_LANES,), 1.0, jnp.float32)

        plsc.subcore_barrier()

        ebase = (c * _NS + s) * ept

        def drain_out(b):
            pltpu.make_async_copy(rel_hbm.at[pl.ds(ebase, K)],
                                  rrows[b], sem_out[b]).wait()
            pltpu.make_async_copy(z1_hbm.at[pl.ds(0, K)],
                                  ones, sem_out[b]).wait()

        def do_chunk(b, base, hin):
            hin[0].wait()
            hin[1].wait()
            g1 = pltpu.async_copy(node_hbm.at[hidx[b]], nrows[b], sem_in[b])
            g2 = pltpu.async_copy(rel_hbm.at[pl.ds(base, K)], rrows[b],
                                  sem_in[b])
            g1.wait()
            g2.wait()
            rr = rrows[b]
            nr = nrows[b]

            @pl.loop(0, K)
            def _(r):
                for j in range(D // _LANES):
                    sl = pl.ds(j * _LANES, _LANES)
                    rr[r, sl] = rr[r, sl] + nr[r, sl]

            pltpu.async_copy(rr, agg_sh.at[tidx[b]], sem_out[b], add=True)
            pltpu.async_copy(ones, deg_sh.at[tidx[b]], sem_out[b], add=True)

        @pl.loop(0, nch // NB)
        def _(o):
            cbase = ebase + o * (NB * K)
            idx_cps = []
            for b in range(NB):
                base = cbase + b * K

                @pl.when(o > 0)
                def _():
                    drain_out(b)

                idx_cps.append((
                    pltpu.async_copy(head_hbm.at[pl.ds(base, K)], hidx[b],
                                     sem_idx[b]),
                    pltpu.async_copy(tail_hbm.at[pl.ds(base, K)], tidx[b],
                                     sem_idx[b]),
                ))
            for b in range(NB):
                do_chunk(b, cbase + b * K, idx_cps[b])

        for t in range(nch % NB):
            base = ebase + (nch - nch % NB + t) * K
            drain_out(t)
            i1 = pltpu.async_copy(head_hbm.at[pl.ds(base, K)], hidx[t],
                                  sem_idx[t])
            i2 = pltpu.async_copy(tail_hbm.at[pl.ds(base, K)], tidx[t],
                                  sem_idx[t])
            do_chunk(t, base, (i1, i2))
        for b in range(NB):
            drain_out(b)

        plsc.subcore_barrier()

        @pl.when(s < nzt)
        def _():
            pltpu.sync_copy(agg_sh.at[pl.ds(s * rpt, rpt)],
                            agg_out.at[c, pl.ds(s * rpt, rpt)])

        @pl.when(s == 0)
        def _():
            pltpu.sync_copy(deg_sh, deg_out.at[c])

    return sc_kernel


def _tc_dense(N, D, B):
    def body(node_ref, aggp_ref, degp_ref, wm_ref, bm_ref, wu1_ref, wu2_ref,
             bu_ref, g_ref, b_ref, out_ref):
        node = node_ref[...]
        agg_pre = aggp_ref[0] + aggp_ref[1]
        deg = degp_ref[:, 0:1] + degp_ref[:, 1:2]

        agg = jnp.dot(agg_pre, wm_ref[...], preferred_element_type=jnp.float32)
        agg = agg + deg * bm_ref[...]
        agg = agg / jnp.maximum(deg, 1.0)

        upd = (jnp.dot(node, wu1_ref[...], preferred_element_type=jnp.float32)
               + jnp.dot(agg, wu2_ref[...], preferred_element_type=jnp.float32)
               + bu_ref[...])
        gelu = 0.5 * upd * (1.0 + lax.erf(upd * 0.7071067811865476))
        out = node + gelu

        mean = jnp.mean(out, axis=-1, keepdims=True)
        cent = out - mean
        var = jnp.mean(cent * cent, axis=-1, keepdims=True)
        normed = cent * lax.rsqrt(var + 1e-5)
        out_ref[...] = normed * g_ref[...] + b_ref[...]

    grid = (N // B,)
    return pl.pallas_call(
        body,
        grid=grid,
        in_specs=[
            pl.BlockSpec((B, D), lambda i: (i, 0)),
            pl.BlockSpec((_NC, B, D), lambda i: (0, i, 0)),
            pl.BlockSpec((B, _NC), lambda i: (i, 0)),
            pl.BlockSpec((D, D), lambda i: (0, 0)),
            pl.BlockSpec((1, D), lambda i: (0, 0)),
            pl.BlockSpec((D, D), lambda i: (0, 0)),
            pl.BlockSpec((D, D), lambda i: (0, 0)),
            pl.BlockSpec((1, D), lambda i: (0, 0)),
            pl.BlockSpec((1, D), lambda i: (0, 0)),
            pl.BlockSpec((1, D), lambda i: (0, 0)),
        ],
        out_specs=pl.BlockSpec((B, D), lambda i: (i, 0)),
        out_shape=jax.ShapeDtypeStruct((N, D), jnp.float32),
    )


def kernel(node_tokens, relation_tokens, edge_index, num_nodes,
           W_msg, b_msg, W_upd, b_upd, gamma, beta):
    N, D = node_tokens.shape
    E = relation_tokens.shape[0]
    K = 80
    NB = 2
    assert E % (_NC * _NS * K) == 0 and N % _NS == 0 and D % _LANES == 0

    head = edge_index[0]
    tail = edge_index[1]
    z2 = jnp.zeros((N, D), jnp.float32)
    z1 = jnp.zeros((N,), jnp.float32)

    agg_parts, deg_parts = _sc_scatter(N, D, E, K, NB)(
        head, tail, node_tokens, relation_tokens, z2, z1)

    deg_t = deg_parts.T
    wm_t = W_msg.T
    wu1_t = W_upd[:, :D].T
    wu2_t = W_upd[:, D:].T

    B = 2000
    out = _tc_dense(N, D, B)(
        node_tokens, agg_parts, deg_t, wm_t,
        b_msg.reshape(1, D), wu1_t, wu2_t,
        b_upd.reshape(1, D), gamma.reshape(1, D), beta.reshape(1, D))
    return out


# speedup vs baseline: 8.0007x; 1.2725x over previous
import functools

import jax
import jax.numpy as jnp
from jax import lax
from jax.experimental import pallas as pl
from jax.experimental.pallas import tpu as pltpu
from jax.experimental.pallas import tpu_sc as plsc

_NC = 2
_NS = 16
_LANES = 16


def _sc_scatter(N, D, E, K, NB):
    NW = _NC * _NS
    ept = E // NW
    nch = ept // K
    rpt = max(8, ((N // _NS + 7) // 8) * 8)
    while N % rpt != 0:
        rpt += 8
    nzt = N // rpt

    mesh = plsc.VectorSubcoreMesh(core_axis_name="c", subcore_axis_name="s")

    @functools.partial(
        pl.kernel,
        out_type=[
            jax.ShapeDtypeStruct((_NC, N, D), jnp.float32),
            jax.ShapeDtypeStruct((_NC, N), jnp.float32),
        ],
        mesh=mesh,
        scratch_types=[
            [pltpu.VMEM((K,), jnp.int32)] * NB,
            [pltpu.VMEM((K,), jnp.int32)] * NB,
            [pltpu.VMEM((K, D), jnp.float32)] * NB,
            [pltpu.VMEM((K, D), jnp.float32)] * NB,
            pltpu.VMEM((K,), jnp.float32),
            pltpu.VMEM_SHARED((N, D), jnp.float32),
            pltpu.VMEM_SHARED((N,), jnp.float32),
            [pltpu.SemaphoreType.DMA] * NB,
            [pltpu.SemaphoreType.DMA] * NB,
            [pltpu.SemaphoreType.DMA] * NB,
        ],
    )
    def sc_kernel(head_hbm, tail_hbm, node_hbm, rel_hbm, z2_hbm, z1_hbm,
                  agg_out, deg_out,
                  hidx, tidx, nrows, rrows, ones, agg_sh, deg_sh,
                  sem_idx, sem_in, sem_out):
        c = lax.axis_index("c")
        s = lax.axis_index("s")

        @pl.when(s < nzt)
        def _():
            pltpu.sync_copy(z2_hbm.at[pl.ds(s * rpt, rpt)],
                            agg_sh.at[pl.ds(s * rpt, rpt)])

        @pl.when(s == 0)
        def _():
            pltpu.sync_copy(z1_hbm, deg_sh)

        @pl.loop(0, K // _LANES)
        def _(j):
            ones[pl.ds(j * _LANES, _LANES)] = jnp.full((_LANES,), 1.0, jnp.float32)

        plsc.subcore_barrier()

        ebase = (c * _NS + s) * ept

        def drain_out(b):
            pltpu.make_async_copy(rel_hbm.at[pl.ds(ebase, K)],
                                  rrows[b], sem_out[b]).wait()
            pltpu.make_async_copy(rel_hbm.at[pl.ds(ebase, K)],
                                  nrows[b], sem_out[b]).wait()
            pltpu.make_async_copy(z1_hbm.at[pl.ds(0, K)],
                                  ones, sem_out[b]).wait()

        def do_chunk(b, base, hin):
            hin[0].wait()
            hin[1].wait()
            g1 = pltpu.async_copy(node_hbm.at[hidx[b]], nrows[b], sem_in[b])
            g2 = pltpu.async_copy(rel_hbm.at[pl.ds(base, K)], rrows[b],
                                  sem_in[b])
            g1.wait()
            pltpu.async_copy(nrows[b], agg_sh.at[tidx[b]], sem_out[b],
                             add=True)
            g2.wait()
            pltpu.async_copy(rrows[b], agg_sh.at[tidx[b]], sem_out[b],
                             add=True)
            pltpu.async_copy(ones, deg_sh.at[tidx[b]], sem_out[b], add=True)

        @pl.loop(0, nch // NB)
        def _(o):
            cbase = ebase + o * (NB * K)
            idx_cps = []
            for b in range(NB):
                base = cbase + b * K

                @pl.when(o > 0)
                def _():
                    drain_out(b)

                idx_cps.append((
                    pltpu.async_copy(head_hbm.at[pl.ds(base, K)], hidx[b],
                                     sem_idx[b]),
                    pltpu.async_copy(tail_hbm.at[pl.ds(base, K)], tidx[b],
                                     sem_idx[b]),
                ))
            for b in range(NB):
                do_chunk(b, cbase + b * K, idx_cps[b])

        for t in range(nch % NB):
            base = ebase + (nch - nch % NB + t) * K
            drain_out(t)
            i1 = pltpu.async_copy(head_hbm.at[pl.ds(base, K)], hidx[t],
                                  sem_idx[t])
            i2 = pltpu.async_copy(tail_hbm.at[pl.ds(base, K)], tidx[t],
                                  sem_idx[t])
            do_chunk(t, base, (i1, i2))
        for b in range(NB):
            drain_out(b)

        plsc.subcore_barrier()

        @pl.when(s < nzt)
        def _():
            pltpu.sync_copy(agg_sh.at[pl.ds(s * rpt, rpt)],
                            agg_out.at[c, pl.ds(s * rpt, rpt)])

        @pl.when(s == 0)
        def _():
            pltpu.sync_copy(deg_sh, deg_out.at[c])

    return sc_kernel


def _tc_dense(N, D, B):
    def body(node_ref, aggp_ref, degp_ref, wm_ref, bm_ref, wu1_ref, wu2_ref,
             bu_ref, g_ref, b_ref, out_ref):
        node = node_ref[...]
        agg_pre = aggp_ref[0] + aggp_ref[1]
        deg = degp_ref[:, 0:1] + degp_ref[:, 1:2]

        agg = jnp.dot(agg_pre, wm_ref[...], preferred_element_type=jnp.float32)
        agg = agg + deg * bm_ref[...]
        agg = agg / jnp.maximum(deg, 1.0)

        upd = (jnp.dot(node, wu1_ref[...], preferred_element_type=jnp.float32)
               + jnp.dot(agg, wu2_ref[...], preferred_element_type=jnp.float32)
               + bu_ref[...])
        gelu = 0.5 * upd * (1.0 + lax.erf(upd * 0.7071067811865476))
        out = node + gelu

        mean = jnp.mean(out, axis=-1, keepdims=True)
        cent = out - mean
        var = jnp.mean(cent * cent, axis=-1, keepdims=True)
        normed = cent * lax.rsqrt(var + 1e-5)
        out_ref[...] = normed * g_ref[...] + b_ref[...]

    grid = (N // B,)
    return pl.pallas_call(
        body,
        grid=grid,
        in_specs=[
            pl.BlockSpec((B, D), lambda i: (i, 0)),
            pl.BlockSpec((_NC, B, D), lambda i: (0, i, 0)),
            pl.BlockSpec((B, _NC), lambda i: (i, 0)),
            pl.BlockSpec((D, D), lambda i: (0, 0)),
            pl.BlockSpec((1, D), lambda i: (0, 0)),
            pl.BlockSpec((D, D), lambda i: (0, 0)),
            pl.BlockSpec((D, D), lambda i: (0, 0)),
            pl.BlockSpec((1, D), lambda i: (0, 0)),
            pl.BlockSpec((1, D), lambda i: (0, 0)),
            pl.BlockSpec((1, D), lambda i: (0, 0)),
        ],
        out_specs=pl.BlockSpec((B, D), lambda i: (i, 0)),
        out_shape=jax.ShapeDtypeStruct((N, D), jnp.float32),
    )


def kernel(node_tokens, relation_tokens, edge_index, num_nodes,
           W_msg, b_msg, W_upd, b_upd, gamma, beta):
    N, D = node_tokens.shape
    E = relation_tokens.shape[0]
    K = 80
    NB = 2
    assert E % (_NC * _NS * K) == 0 and N % _NS == 0 and D % _LANES == 0

    head = edge_index[0]
    tail = edge_index[1]
    z2 = jnp.zeros((N, D), jnp.float32)
    z1 = jnp.zeros((N,), jnp.float32)

    agg_parts, deg_parts = _sc_scatter(N, D, E, K, NB)(
        head, tail, node_tokens, relation_tokens, z2, z1)

    deg_t = deg_parts.T
    wm_t = W_msg.T
    wu1_t = W_upd[:, :D].T
    wu2_t = W_upd[:, D:].T

    B = 2000
    out = _tc_dense(N, D, B)(
        node_tokens, agg_parts, deg_t, wm_t,
        b_msg.reshape(1, D), wu1_t, wu2_t,
        b_upd.reshape(1, D), gamma.reshape(1, D), beta.reshape(1, D))
    return out


# speedup vs baseline: 10.2502x; 1.2812x over previous
import functools

import jax
import jax.numpy as jnp
from jax import lax
from jax.experimental import pallas as pl
from jax.experimental.pallas import tpu as pltpu
from jax.experimental.pallas import tpu_sc as plsc

_NC = 2
_NS = 16
_LANES = 16


def _sc_scatter(N, D, E, K, NB):
    NW = _NC * _NS
    ept = E // NW
    nch = ept // K
    rpt = max(8, ((N // _NS + 7) // 8) * 8)
    while N % rpt != 0:
        rpt += 8
    nzt = N // rpt

    mesh = plsc.VectorSubcoreMesh(core_axis_name="c", subcore_axis_name="s")

    @functools.partial(
        pl.kernel,
        out_type=[
            jax.ShapeDtypeStruct((_NC, N, D), jnp.float32),
            jax.ShapeDtypeStruct((_NC, N), jnp.float32),
        ],
        mesh=mesh,
        scratch_types=[
            [pltpu.VMEM((K,), jnp.int32)] * NB,
            [pltpu.VMEM((K,), jnp.int32)] * NB,
            [pltpu.VMEM((K, D), jnp.float32)] * NB,
            [pltpu.VMEM((K, D), jnp.float32)] * NB,
            pltpu.VMEM((K,), jnp.float32),
            pltpu.VMEM_SHARED((N, D), jnp.float32),
            pltpu.VMEM_SHARED((N,), jnp.float32),
            [pltpu.SemaphoreType.DMA] * NB,
            [pltpu.SemaphoreType.DMA] * NB,
            [pltpu.SemaphoreType.DMA] * NB,
        ],
    )
    def sc_kernel(head_hbm, tail_hbm, node_hbm, rel_hbm, z2_hbm, z1_hbm,
                  agg_out, deg_out,
                  hidx, tidx, nrows, rrows, ones, agg_sh, deg_sh,
                  sem_idx, sem_in, sem_out):
        c = lax.axis_index("c")
        s = lax.axis_index("s")

        @pl.when(s < nzt)
        def _():
            pltpu.sync_copy(z2_hbm.at[pl.ds(s * rpt, rpt)],
                            agg_sh.at[pl.ds(s * rpt, rpt)])

        @pl.when(s == 0)
        def _():
            pltpu.sync_copy(z1_hbm, deg_sh)

        @pl.loop(0, K // _LANES)
        def _(j):
            ones[pl.ds(j * _LANES, _LANES)] = jnp.full((_LANES,), 1.0, jnp.float32)

        plsc.subcore_barrier()

        ebase = (c * _NS + s) * ept
        nfull = ept // K

        def issue_idx(b, ci):
            base = ebase + ci * K
            pltpu.async_copy(head_hbm.at[pl.ds(base, K)], hidx[b], sem_idx[b])
            pltpu.async_copy(tail_hbm.at[pl.ds(base, K)], tidx[b], sem_idx[b])

        def wait_idx(b):
            pltpu.make_async_copy(head_hbm.at[pl.ds(ebase, K)], hidx[b],
                                  sem_idx[b]).wait()
            pltpu.make_async_copy(head_hbm.at[pl.ds(ebase, K)], tidx[b],
                                  sem_idx[b]).wait()

        def issue_in(b, ci):
            base = ebase + ci * K
            pltpu.async_copy(node_hbm.at[hidx[b]], nrows[b], sem_in[b])
            pltpu.async_copy(rel_hbm.at[pl.ds(base, K)], rrows[b], sem_in[b])

        def finish(b):
            pltpu.make_async_copy(rel_hbm.at[pl.ds(ebase, K)], nrows[b],
                                  sem_in[b]).wait()
            pltpu.make_async_copy(rel_hbm.at[pl.ds(ebase, K)], rrows[b],
                                  sem_in[b]).wait()
            pltpu.async_copy(nrows[b], agg_sh.at[tidx[b]], sem_out[b],
                             add=True)
            pltpu.async_copy(rrows[b], agg_sh.at[tidx[b]], sem_out[b],
                             add=True)
            pltpu.async_copy(ones, deg_sh.at[tidx[b]], sem_out[b], add=True)

        def drain_out(b):
            pltpu.make_async_copy(rel_hbm.at[pl.ds(ebase, K)], nrows[b],
                                  sem_out[b]).wait()
            pltpu.make_async_copy(rel_hbm.at[pl.ds(ebase, K)], rrows[b],
                                  sem_out[b]).wait()
            pltpu.make_async_copy(z1_hbm.at[pl.ds(0, K)], ones,
                                  sem_out[b]).wait()

        def step(ci, b):
            bn = (b + 1) % NB
            wait_idx(b)
            issue_in(b, ci)

            @pl.when(ci >= NB - 1)
            def _():
                drain_out(bn)

            @pl.when(ci + 1 < nfull)
            def _():
                issue_idx(bn, ci + 1)

            @pl.when(ci >= 1)
            def _():
                finish((b + NB - 1) % NB)

        issue_idx(0, 0)
        rem = nfull % NB

        @pl.loop(0, nfull // NB)
        def _(o):
            for db in range(NB):
                step(o * NB + db, db)

        for t in range(rem):
            ci = nfull - rem + t
            step(ci, ci % NB)
        b_last = (nfull - 1) % NB
        finish(b_last)
        for db in range(2, NB + 1):
            drain_out((b_last + db) % NB)

        plsc.subcore_barrier()

        @pl.when(s < nzt)
        def _():
            pltpu.sync_copy(agg_sh.at[pl.ds(s * rpt, rpt)],
                            agg_out.at[c, pl.ds(s * rpt, rpt)])

        @pl.when(s == 0)
        def _():
            pltpu.sync_copy(deg_sh, deg_out.at[c])

    return sc_kernel


def _tc_dense(N, D, B):
    def body(node_ref, aggp_ref, degp_ref, wm_ref, bm_ref, wu1_ref, wu2_ref,
             bu_ref, g_ref, b_ref, out_ref):
        node = node_ref[...]
        agg_pre = aggp_ref[0] + aggp_ref[1]
        deg = degp_ref[:, 0:1] + degp_ref[:, 1:2]

        agg = jnp.dot(agg_pre, wm_ref[...], preferred_element_type=jnp.float32)
        agg = agg + deg * bm_ref[...]
        agg = agg / jnp.maximum(deg, 1.0)

        upd = (jnp.dot(node, wu1_ref[...], preferred_element_type=jnp.float32)
               + jnp.dot(agg, wu2_ref[...], preferred_element_type=jnp.float32)
               + bu_ref[...])
        gelu = 0.5 * upd * (1.0 + lax.erf(upd * 0.7071067811865476))
        out = node + gelu

        mean = jnp.mean(out, axis=-1, keepdims=True)
        cent = out - mean
        var = jnp.mean(cent * cent, axis=-1, keepdims=True)
        normed = cent * lax.rsqrt(var + 1e-5)
        out_ref[...] = normed * g_ref[...] + b_ref[...]

    grid = (N // B,)
    return pl.pallas_call(
        body,
        grid=grid,
        in_specs=[
            pl.BlockSpec((B, D), lambda i: (i, 0)),
            pl.BlockSpec((_NC, B, D), lambda i: (0, i, 0)),
            pl.BlockSpec((B, _NC), lambda i: (i, 0)),
            pl.BlockSpec((D, D), lambda i: (0, 0)),
            pl.BlockSpec((1, D), lambda i: (0, 0)),
            pl.BlockSpec((D, D), lambda i: (0, 0)),
            pl.BlockSpec((D, D), lambda i: (0, 0)),
            pl.BlockSpec((1, D), lambda i: (0, 0)),
            pl.BlockSpec((1, D), lambda i: (0, 0)),
            pl.BlockSpec((1, D), lambda i: (0, 0)),
        ],
        out_specs=pl.BlockSpec((B, D), lambda i: (i, 0)),
        out_shape=jax.ShapeDtypeStruct((N, D), jnp.float32),
    )


def kernel(node_tokens, relation_tokens, edge_index, num_nodes,
           W_msg, b_msg, W_upd, b_upd, gamma, beta):
    N, D = node_tokens.shape
    E = relation_tokens.shape[0]
    K = 40
    NB = 4
    assert E % (_NC * _NS * K) == 0 and N % _NS == 0 and D % _LANES == 0

    head = edge_index[0]
    tail = edge_index[1]
    z2 = jnp.zeros((N, D), jnp.float32)
    z1 = jnp.zeros((N,), jnp.float32)

    agg_parts, deg_parts = _sc_scatter(N, D, E, K, NB)(
        head, tail, node_tokens, relation_tokens, z2, z1)

    deg_t = deg_parts.T
    wm_t = W_msg.T
    wu1_t = W_upd[:, :D].T
    wu2_t = W_upd[:, D:].T

    B = 2000
    out = _tc_dense(N, D, B)(
        node_tokens, agg_parts, deg_t, wm_t,
        b_msg.reshape(1, D), wu1_t, wu2_t,
        b_upd.reshape(1, D), gamma.reshape(1, D), beta.reshape(1, D))
    return out
